# baseline (speedup 1.0000x reference)
import functools

import jax
import jax.numpy as jnp
from jax import lax
from jax.experimental import pallas as pl
from jax.experimental.pallas import tpu as pltpu
from jax.experimental.pallas import tpu_sc as plsc

N = 10000
NPAD = 10240
E = 320000
EPAD = 327680
NB = 2560
NC = 2
NS = 16
NW = NC * NS
BPW = NB // NW
RPS = NPAD // NS
HB = NPAD // 128

_mesh = plsc.VectorSubcoreMesh(core_axis_name="c", subcore_axis_name="s")


@functools.partial(
    pl.kernel,
    out_type=jax.ShapeDtypeStruct((NC, HB, 128), jnp.float32),
    mesh=_mesh,
    scratch_types=[
        pltpu.VMEM((BPW, 128), jnp.int32),
        pltpu.VMEM((HB, 128), jnp.float32),
        pltpu.VMEM((HB,), jnp.int32),
        pltpu.VMEM_SHARED((HB, 128), jnp.float32),
    ],
    compiler_params=pltpu.CompilerParams(needs_layout_passes=False),
)
def _deg_pass(dst_hbm, zeros_hbm, idlist_hbm, out_hbm,
              dst_idx, hist, idlist, acc):
    c = lax.axis_index("c")
    s = lax.axis_index("s")
    w = c * NS + s
    pltpu.sync_copy(dst_hbm.at[pl.ds(w * BPW, BPW)], dst_idx)
    pltpu.sync_copy(idlist_hbm, idlist)

    def zbody(i, _):
        for j in range(8):
            hist[i, pl.ds(j * 16, 16)] = jnp.zeros((16,), jnp.float32)
        return 0

    lax.fori_loop(0, HB, zbody, 0)

    @pl.when(s < 10)
    def _():
        pltpu.sync_copy(zeros_hbm, acc.at[pl.ds(s * 8, 8)])

    plsc.subcore_barrier()

    def body(b, _):
        for j in range(8):
            iv = dst_idx[b, pl.ds(j * 16, 16)]
            hi = lax.shift_right_logical(iv, 7)
            lo = lax.bitwise_and(iv, 127)
            cnt, last = plsc.scan_count(iv)
            cur = plsc.load_gather(hist, (hi, lo))
            plsc.store_scatter(hist, (hi, lo),
                               cur + cnt.astype(jnp.float32), mask=last)
        return 0

    lax.fori_loop(0, BPW, body, 0)

    pltpu.sync_copy(hist, acc.at[idlist], add=True)
    plsc.subcore_barrier()

    @pl.when(s < 10)
    def _():
        pltpu.sync_copy(acc.at[pl.ds(s * 8, 8)],
                        out_hbm.at[c, pl.ds(s * 8, 8)])


@functools.partial(
    pl.kernel,
    out_type=jax.ShapeDtypeStruct((NC, NPAD, 128), jnp.float32),
    mesh=_mesh,
    scratch_types=[
        pltpu.VMEM((BPW // 2, 128), jnp.int32),
        pltpu.VMEM((BPW // 2, 128), jnp.int32),
        pltpu.VMEM((128, 128), jnp.float32),
        pltpu.VMEM((128, 128), jnp.float32),
        pltpu.VMEM_SHARED((NPAD, 128), jnp.float32),
        pltpu.SemaphoreType.DMA,
        pltpu.SemaphoreType.DMA,
        pltpu.SemaphoreType.DMA,
        pltpu.SemaphoreType.DMA,
    ],
)
def _agg_pass(g_hbm, src_hbm, dst_hbm, zeros_hbm, out_hbm,
              src_idx, dst_idx, rows0, rows1, acc, sem0, sem1, ssem0, ssem1):
    c = lax.axis_index("c")
    s = lax.axis_index("s")
    w = c * NS + s
    pltpu.sync_copy(zeros_hbm, acc.at[pl.ds(s * RPS, RPS)])
    plsc.subcore_barrier()

    HALF = BPW // 2

    def _gwait(buf, sem):
        pltpu.make_async_copy(g_hbm.at[pl.ds(0, 128)], buf, sem).wait()

    for h in range(2):
        base = w * BPW + h * HALF
        pltpu.sync_copy(src_hbm.at[pl.ds(base, HALF)], src_idx)
        pltpu.sync_copy(dst_hbm.at[pl.ds(base, HALF)], dst_idx)
        pltpu.async_copy(g_hbm.at[src_idx.at[0]], rows0, sem0)

        def body(b2, _):
            b = 2 * b2
            _gwait(rows0, sem0)

            @pl.when(b2 > 0)
            def _():
                _gwait(rows1, ssem1)

            pltpu.async_copy(g_hbm.at[src_idx.at[b + 1]], rows1, sem1)
            pltpu.async_copy(rows0, acc.at[dst_idx.at[b]], ssem0, add=True)
            _gwait(rows1, sem1)

            @pl.when(b2 < HALF // 2 - 1)
            def _():
                _gwait(rows0, ssem0)
                pltpu.async_copy(g_hbm.at[src_idx.at[b + 2]], rows0, sem0)

            pltpu.async_copy(rows1, acc.at[dst_idx.at[b + 1]], ssem1, add=True)
            return 0

        lax.fori_loop(0, HALF // 2, body, 0)
        _gwait(rows0, ssem0)
        _gwait(rows1, ssem1)
    plsc.subcore_barrier()
    pltpu.sync_copy(acc.at[pl.ds(s * RPS, RPS)],
                    out_hbm.at[c, pl.ds(s * RPS, RPS)])


def _dinv_of(degp_blk):
    d = degp_blk[0, 0:1] + degp_blk[0, 1:2] + 1.0
    dinv = lax.rsqrt(d)
    ones = jnp.ones((1, 128), jnp.float32)
    return lax.dot_general(dinv, ones, (((0,), (0,)), ((), ())),
                           preferred_element_type=jnp.float32)


def _k1_body(x_ref, w_ref, degp_ref, g_ref):
    h = jnp.dot(x_ref[...], w_ref[...], preferred_element_type=jnp.float32)
    g_ref[...] = h * _dinv_of(degp_ref[...])


def _k2_body(p_ref, g1_ref, degp_ref, b1_ref, wcat_ref, g2_ref):
    dinv = _dinv_of(degp_ref[...])
    t = (p_ref[0] + p_ref[1] + g1_ref[...]) * dinv
    h = jnp.maximum(t + b1_ref[...], 0.0)
    g2 = jnp.dot(h, wcat_ref[...], preferred_element_type=jnp.float32) * dinv
    row = pl.program_id(0) * 128 + lax.broadcasted_iota(jnp.int32, (128, 1), 0)
    g2_ref[...] = jnp.where(row < N, g2, 0.0)


def _k3_body(p_ref, g2_ref, degp_ref, bcat_ref, o_ref):
    dinv = _dinv_of(degp_ref[...])
    o_ref[...] = (p_ref[0] + p_ref[1] + g2_ref[...]) * dinv + bcat_ref[...]


_spec_rows = pl.BlockSpec((128, 128), lambda i: (i, 0))
_spec_w = pl.BlockSpec((128, 128), lambda i: (0, 0))
_spec_degp = pl.BlockSpec((1, 2, 128), lambda i: (i, 0, 0))
_spec_p = pl.BlockSpec((2, 128, 128), lambda i: (0, i, 0))
_spec_b = pl.BlockSpec((1, 128), lambda i: (0, 0))

_k1 = pl.pallas_call(
    _k1_body,
    grid=(NPAD // 128,),
    in_specs=[_spec_rows, _spec_w, _spec_degp],
    out_specs=_spec_rows,
    out_shape=jax.ShapeDtypeStruct((NPAD, 128), jnp.float32),
)

_k2 = pl.pallas_call(
    _k2_body,
    grid=(NPAD // 128,),
    in_specs=[_spec_p, _spec_rows, _spec_degp, _spec_b, _spec_w],
    out_specs=_spec_rows,
    out_shape=jax.ShapeDtypeStruct((NPAD, 128), jnp.float32),
)

_k3 = pl.pallas_call(
    _k3_body,
    grid=(NPAD // 128,),
    in_specs=[_spec_p, _spec_rows, _spec_degp, _spec_b],
    out_specs=_spec_rows,
    out_shape=jax.ShapeDtypeStruct((NPAD, 128), jnp.float32),
)


def kernel(x, edge_index, W1, b1, Wmu, bmu, Wls, bls):
    e = edge_index.astype(jnp.int32)
    pad = N + jnp.arange(EPAD - E, dtype=jnp.int32) % (NPAD - N)
    src2d = jnp.concatenate([e[0], pad]).reshape(NB, 128)
    dst2d = jnp.concatenate([e[1], pad]).reshape(NB, 128)

    x_pad = jnp.pad(x, ((0, NPAD - N), (0, 0)))
    wcat = jnp.concatenate([Wmu, Wls], axis=1)
    bcat = jnp.concatenate([bmu, bls]).reshape(1, 128)
    b1r = b1.reshape(1, 128)

    zeros8 = jnp.zeros((8, 128), jnp.float32)
    idlist = jnp.arange(HB, dtype=jnp.int32)
    zeros128 = jnp.zeros((RPS, 128), jnp.float32)

    degp = _deg_pass(dst2d, zeros8, idlist)
    degp_t = jnp.swapaxes(degp, 0, 1)
    g1 = _k1(x_pad, W1, degp_t)
    p1 = _agg_pass(g1, src2d, dst2d, zeros128)
    g2 = _k2(p1, g1, degp_t, b1r, wcat)
    p2 = _agg_pass(g2, src2d, dst2d, zeros128)
    out = _k3(p2, g2, degp_t, bcat)
    return (out[:N, :64], out[:N, 64:])


# speedup vs baseline: 1.0008x; 1.0008x over previous
import functools

import jax
import jax.numpy as jnp
from jax import lax
from jax.experimental import pallas as pl
from jax.experimental.pallas import tpu as pltpu
from jax.experimental.pallas import tpu_sc as plsc

N = 10000
NPAD = 10240
E = 320000
EPAD = 327680
NB = 2560
NC = 2
NS = 16
NW = NC * NS
BPW = NB // NW
RPS = NPAD // NS
HB = NPAD // 128

_mesh = plsc.VectorSubcoreMesh(core_axis_name="c", subcore_axis_name="s")


@functools.partial(
    pl.kernel,
    out_type=jax.ShapeDtypeStruct((NC, HB, 128), jnp.float32),
    mesh=_mesh,
    scratch_types=[
        pltpu.VMEM((BPW, 128), jnp.int32),
        pltpu.VMEM((HB, 128), jnp.float32),
        pltpu.VMEM((HB,), jnp.int32),
        pltpu.VMEM_SHARED((HB, 128), jnp.float32),
    ],
    compiler_params=pltpu.CompilerParams(needs_layout_passes=False),
)
def _deg_pass(dst_hbm, zeros_hbm, idlist_hbm, out_hbm,
              dst_idx, hist, idlist, acc):
    c = lax.axis_index("c")
    s = lax.axis_index("s")
    w = c * NS + s
    pltpu.sync_copy(dst_hbm.at[pl.ds(w * BPW, BPW)], dst_idx)
    pltpu.sync_copy(idlist_hbm, idlist)

    def zbody(i, _):
        for j in range(8):
            hist[i, pl.ds(j * 16, 16)] = jnp.zeros((16,), jnp.float32)
        return 0

    lax.fori_loop(0, HB, zbody, 0)

    @pl.when(s < 10)
    def _():
        pltpu.sync_copy(zeros_hbm, acc.at[pl.ds(s * 8, 8)])

    plsc.subcore_barrier()

    def body(b, _):
        for j in range(8):
            iv = dst_idx[b, pl.ds(j * 16, 16)]
            hi = lax.shift_right_logical(iv, 7)
            lo = lax.bitwise_and(iv, 127)
            cnt, last = plsc.scan_count(iv)
            cur = plsc.load_gather(hist, (hi, lo))
            plsc.store_scatter(hist, (hi, lo),
                               cur + cnt.astype(jnp.float32), mask=last)
        return 0

    lax.fori_loop(0, BPW, body, 0)

    pltpu.sync_copy(hist, acc.at[idlist], add=True)
    plsc.subcore_barrier()

    @pl.when(s < 10)
    def _():
        pltpu.sync_copy(acc.at[pl.ds(s * 8, 8)],
                        out_hbm.at[c, pl.ds(s * 8, 8)])


@functools.partial(
    pl.kernel,
    out_type=jax.ShapeDtypeStruct((NC, NPAD, 128), jnp.float32),
    mesh=_mesh,
    scratch_types=[
        pltpu.VMEM((BPW // 2, 128), jnp.int32),
        pltpu.VMEM((BPW // 2, 128), jnp.int32),
        pltpu.VMEM((128, 128), jnp.float32),
        pltpu.VMEM((128, 128), jnp.float32),
        pltpu.VMEM_SHARED((NPAD, 128), jnp.float32),
        pltpu.SemaphoreType.DMA,
        pltpu.SemaphoreType.DMA,
    ],
)
def _agg_pass(g_hbm, src_hbm, dst_hbm, zeros_hbm, out_hbm,
              src_idx, dst_idx, rows0, rows1, acc, sem0, sem1):
    c = lax.axis_index("c")
    s = lax.axis_index("s")
    w = c * NS + s
    pltpu.sync_copy(zeros_hbm, acc.at[pl.ds(s * RPS, RPS)])
    plsc.subcore_barrier()

    HALF = BPW // 2
    for h in range(2):
        base = w * BPW + h * HALF
        pltpu.sync_copy(src_hbm.at[pl.ds(base, HALF)], src_idx)
        pltpu.sync_copy(dst_hbm.at[pl.ds(base, HALF)], dst_idx)
        pltpu.async_copy(g_hbm.at[src_idx.at[0]], rows0, sem0)

        def body(b2, _):
            b = 2 * b2
            pltpu.make_async_copy(g_hbm.at[pl.ds(0, 128)], rows0, sem0).wait()
            pltpu.async_copy(g_hbm.at[src_idx.at[b + 1]], rows1, sem1)
            pltpu.sync_copy(rows0, acc.at[dst_idx.at[b]], add=True)
            pltpu.make_async_copy(g_hbm.at[pl.ds(0, 128)], rows1, sem1).wait()

            @pl.when(b2 < HALF // 2 - 1)
            def _():
                pltpu.async_copy(g_hbm.at[src_idx.at[b + 2]], rows0, sem0)

            pltpu.sync_copy(rows1, acc.at[dst_idx.at[b + 1]], add=True)
            return 0

        lax.fori_loop(0, HALF // 2, body, 0)
    plsc.subcore_barrier()
    pltpu.sync_copy(acc.at[pl.ds(s * RPS, RPS)],
                    out_hbm.at[c, pl.ds(s * RPS, RPS)])


def _dinv_of(degp_blk):
    d = degp_blk[0, 0:1] + degp_blk[0, 1:2] + 1.0
    dinv = lax.rsqrt(d)
    ones = jnp.ones((1, 128), jnp.float32)
    return lax.dot_general(dinv, ones, (((0,), (0,)), ((), ())),
                           preferred_element_type=jnp.float32)


def _k1_body(x_ref, w_ref, degp_ref, g_ref):
    h = jnp.dot(x_ref[...], w_ref[...], preferred_element_type=jnp.float32)
    g_ref[...] = h * _dinv_of(degp_ref[...])


def _k2_body(p_ref, g1_ref, degp_ref, b1_ref, wcat_ref, g2_ref):
    dinv = _dinv_of(degp_ref[...])
    t = (p_ref[0] + p_ref[1] + g1_ref[...]) * dinv
    h = jnp.maximum(t + b1_ref[...], 0.0)
    g2 = jnp.dot(h, wcat_ref[...], preferred_element_type=jnp.float32) * dinv
    row = pl.program_id(0) * 128 + lax.broadcasted_iota(jnp.int32, (128, 1), 0)
    g2_ref[...] = jnp.where(row < N, g2, 0.0)


def _k3_body(p_ref, g2_ref, degp_ref, bcat_ref, o_ref):
    dinv = _dinv_of(degp_ref[...])
    o_ref[...] = (p_ref[0] + p_ref[1] + g2_ref[...]) * dinv + bcat_ref[...]


_spec_rows = pl.BlockSpec((128, 128), lambda i: (i, 0))
_spec_w = pl.BlockSpec((128, 128), lambda i: (0, 0))
_spec_degp = pl.BlockSpec((1, 2, 128), lambda i: (i, 0, 0))
_spec_p = pl.BlockSpec((2, 128, 128), lambda i: (0, i, 0))
_spec_b = pl.BlockSpec((1, 128), lambda i: (0, 0))

_k1 = pl.pallas_call(
    _k1_body,
    grid=(NPAD // 128,),
    in_specs=[_spec_rows, _spec_w, _spec_degp],
    out_specs=_spec_rows,
    out_shape=jax.ShapeDtypeStruct((NPAD, 128), jnp.float32),
)

_k2 = pl.pallas_call(
    _k2_body,
    grid=(NPAD // 128,),
    in_specs=[_spec_p, _spec_rows, _spec_degp, _spec_b, _spec_w],
    out_specs=_spec_rows,
    out_shape=jax.ShapeDtypeStruct((NPAD, 128), jnp.float32),
)

_k3 = pl.pallas_call(
    _k3_body,
    grid=(NPAD // 128,),
    in_specs=[_spec_p, _spec_rows, _spec_degp, _spec_b],
    out_specs=_spec_rows,
    out_shape=jax.ShapeDtypeStruct((NPAD, 128), jnp.float32),
)


def kernel(x, edge_index, W1, b1, Wmu, bmu, Wls, bls):
    e = edge_index.astype(jnp.int32)
    pad = N + jnp.arange(EPAD - E, dtype=jnp.int32) % (NPAD - N)
    src2d = jnp.concatenate([e[0], pad]).reshape(NB, 128)
    dst2d = jnp.concatenate([e[1], pad]).reshape(NB, 128)

    x_pad = jnp.pad(x, ((0, NPAD - N), (0, 0)))
    wcat = jnp.concatenate([Wmu, Wls], axis=1)
    bcat = jnp.concatenate([bmu, bls]).reshape(1, 128)
    b1r = b1.reshape(1, 128)

    zeros8 = jnp.zeros((8, 128), jnp.float32)
    idlist = jnp.arange(HB, dtype=jnp.int32)
    zeros128 = jnp.zeros((RPS, 128), jnp.float32)

    degp = _deg_pass(dst2d, zeros8, idlist)
    degp_t = jnp.swapaxes(degp, 0, 1)
    g1 = _k1(x_pad, W1, degp_t)
    p1 = _agg_pass(g1, src2d, dst2d, zeros128)
    g2 = _k2(p1, g1, degp_t, b1r, wcat)
    p2 = _agg_pass(g2, src2d, dst2d, zeros128)
    out = _k3(p2, g2, degp_t, bcat)
    return (out[:N, :64], out[:N, 64:])


# speedup vs baseline: 1.0241x; 1.0233x over previous
import functools

import jax
import jax.numpy as jnp
from jax import lax
from jax.experimental import pallas as pl
from jax.experimental.pallas import tpu as pltpu
from jax.experimental.pallas import tpu_sc as plsc

N = 10000
NPAD = 10240
E = 320000
EPAD = 327680
NB = 2560
NC = 2
NS = 16
NW = NC * NS
BPW = NB // NW
RPS = NPAD // NS
HB = NPAD // 128

_mesh = plsc.VectorSubcoreMesh(core_axis_name="c", subcore_axis_name="s")


@functools.partial(
    pl.kernel,
    out_type=jax.ShapeDtypeStruct((NC, HB, 128), jnp.float32),
    mesh=_mesh,
    scratch_types=[
        pltpu.VMEM((BPW, 128), jnp.int32),
        pltpu.VMEM((HB, 128), jnp.float32),
        pltpu.VMEM((HB,), jnp.int32),
        pltpu.VMEM_SHARED((HB, 128), jnp.float32),
    ],
    compiler_params=pltpu.CompilerParams(needs_layout_passes=False),
)
def _deg_pass(dst_hbm, zeros_hbm, idlist_hbm, out_hbm,
              dst_idx, hist, idlist, acc):
    c = lax.axis_index("c")
    s = lax.axis_index("s")
    w = c * NS + s
    pltpu.sync_copy(dst_hbm.at[pl.ds(w * BPW, BPW)], dst_idx)
    pltpu.sync_copy(idlist_hbm, idlist)

    def zbody(i, _):
        for j in range(8):
            hist[i, pl.ds(j * 16, 16)] = jnp.zeros((16,), jnp.float32)
        return 0

    lax.fori_loop(0, HB, zbody, 0)

    @pl.when(s < 10)
    def _():
        pltpu.sync_copy(zeros_hbm, acc.at[pl.ds(s * 8, 8)])

    plsc.subcore_barrier()

    def body(b, _):
        for j in range(8):
            iv = dst_idx[b, pl.ds(j * 16, 16)]
            hi = lax.shift_right_logical(iv, 7)
            lo = lax.bitwise_and(iv, 127)
            cnt, last = plsc.scan_count(iv)
            cur = plsc.load_gather(hist, (hi, lo))
            plsc.store_scatter(hist, (hi, lo),
                               cur + cnt.astype(jnp.float32), mask=last)
        return 0

    lax.fori_loop(0, BPW, body, 0)

    pltpu.sync_copy(hist, acc.at[idlist], add=True)
    plsc.subcore_barrier()

    @pl.when(s < 10)
    def _():
        pltpu.sync_copy(acc.at[pl.ds(s * 8, 8)],
                        out_hbm.at[c, pl.ds(s * 8, 8)])


@functools.partial(
    pl.kernel,
    out_type=jax.ShapeDtypeStruct((NC, NPAD, 128), jnp.float32),
    mesh=_mesh,
    scratch_types=[
        pltpu.VMEM((BPW // 2, 128), jnp.int32),
        pltpu.VMEM((BPW // 2, 128), jnp.int32),
        pltpu.VMEM((128, 128), jnp.float32),
        pltpu.VMEM((128, 128), jnp.float32),
        pltpu.VMEM_SHARED((NPAD, 128), jnp.float32),
        pltpu.SemaphoreType.DMA,
        pltpu.SemaphoreType.DMA,
    ],
)
def _agg_pass(g_hbm, src_hbm, dst_hbm, zeros_hbm, out_hbm,
              src_idx, dst_idx, rows0, rows1, acc, sem0, sem1):
    c = lax.axis_index("c")
    s = lax.axis_index("s")
    w = c * NS + s
    pltpu.sync_copy(zeros_hbm, acc.at[pl.ds(s * RPS, RPS)])
    plsc.subcore_barrier()

    HALF = BPW // 2
    for h in range(2):
        base = w * BPW + h * HALF
        pltpu.sync_copy(src_hbm.at[pl.ds(base, HALF)], src_idx)
        pltpu.sync_copy(dst_hbm.at[pl.ds(base, HALF)], dst_idx)
        pltpu.async_copy(g_hbm.at[src_idx.at[0]], rows0, sem0)

        def body(b2, _):
            b = 2 * b2
            pltpu.make_async_copy(g_hbm.at[pl.ds(0, 128)], rows0, sem0).wait()
            pltpu.async_copy(g_hbm.at[src_idx.at[b + 1]], rows1, sem1)
            pltpu.sync_copy(rows0, acc.at[dst_idx.at[b]], add=True)
            pltpu.make_async_copy(g_hbm.at[pl.ds(0, 128)], rows1, sem1).wait()

            @pl.when(b2 < HALF // 2 - 1)
            def _():
                pltpu.async_copy(g_hbm.at[src_idx.at[b + 2]], rows0, sem0)

            pltpu.sync_copy(rows1, acc.at[dst_idx.at[b + 1]], add=True)
            return 0

        lax.fori_loop(0, HALF // 2, body, 0)
    plsc.subcore_barrier()
    pltpu.sync_copy(acc.at[pl.ds(s * RPS, RPS)],
                    out_hbm.at[c, pl.ds(s * RPS, RPS)])


def _dinv_of(degp_blk):
    d = degp_blk[0, 0] + degp_blk[0, 1] + 1.0
    return lax.rsqrt(d)[:, None]


def _k1_body(x_ref, w_ref, degp_ref, g_ref):
    h = jnp.dot(x_ref[...], w_ref[...], preferred_element_type=jnp.float32)
    g_ref[...] = h * _dinv_of(degp_ref[...])


def _k2_body(p_ref, g1_ref, degp_ref, b1_ref, wcat_ref, g2_ref):
    dinv = _dinv_of(degp_ref[...])
    t = (p_ref[0] + p_ref[1] + g1_ref[...]) * dinv
    h = jnp.maximum(t + b1_ref[...], 0.0)
    g2 = jnp.dot(h, wcat_ref[...], preferred_element_type=jnp.float32) * dinv
    row = pl.program_id(0) * 128 + lax.broadcasted_iota(jnp.int32, (128, 1), 0)
    g2_ref[...] = jnp.where(row < N, g2, 0.0)


def _k3_body(p_ref, g2_ref, degp_ref, bcat_ref, o_ref):
    dinv = _dinv_of(degp_ref[...])
    o_ref[...] = (p_ref[0] + p_ref[1] + g2_ref[...]) * dinv + bcat_ref[...]


_spec_rows = pl.BlockSpec((128, 128), lambda i: (i, 0))
_spec_w = pl.BlockSpec((128, 128), lambda i: (0, 0))
_spec_degp = pl.BlockSpec((1, 2, 128), lambda i: (i, 0, 0))
_spec_p = pl.BlockSpec((2, 128, 128), lambda i: (0, i, 0))
_spec_b = pl.BlockSpec((1, 128), lambda i: (0, 0))

_k1 = pl.pallas_call(
    _k1_body,
    grid=(NPAD // 128,),
    in_specs=[_spec_rows, _spec_w, _spec_degp],
    out_specs=_spec_rows,
    out_shape=jax.ShapeDtypeStruct((NPAD, 128), jnp.float32),
)

_k2 = pl.pallas_call(
    _k2_body,
    grid=(NPAD // 128,),
    in_specs=[_spec_p, _spec_rows, _spec_degp, _spec_b, _spec_w],
    out_specs=_spec_rows,
    out_shape=jax.ShapeDtypeStruct((NPAD, 128), jnp.float32),
)

_k3 = pl.pallas_call(
    _k3_body,
    grid=(NPAD // 128,),
    in_specs=[_spec_p, _spec_rows, _spec_degp, _spec_b],
    out_specs=_spec_rows,
    out_shape=jax.ShapeDtypeStruct((NPAD, 128), jnp.float32),
)


def kernel(x, edge_index, W1, b1, Wmu, bmu, Wls, bls):
    e = edge_index.astype(jnp.int32)
    pad = N + jnp.arange(EPAD - E, dtype=jnp.int32) % (NPAD - N)
    src2d = jnp.concatenate([e[0], pad]).reshape(NB, 128)
    dst2d = jnp.concatenate([e[1], pad]).reshape(NB, 128)

    x_pad = jnp.pad(x, ((0, NPAD - N), (0, 0)))
    wcat = jnp.concatenate([Wmu, Wls], axis=1)
    bcat = jnp.concatenate([bmu, bls]).reshape(1, 128)
    b1r = b1.reshape(1, 128)

    zeros8 = jnp.zeros((8, 128), jnp.float32)
    idlist = jnp.arange(HB, dtype=jnp.int32)
    zeros128 = jnp.zeros((RPS, 128), jnp.float32)

    degp = _deg_pass(dst2d, zeros8, idlist)
    degp_t = jnp.swapaxes(degp, 0, 1)
    g1 = _k1(x_pad, W1, degp_t)
    p1 = _agg_pass(g1, src2d, dst2d, zeros128)
    g2 = _k2(p1, g1, degp_t, b1r, wcat)
    p2 = _agg_pass(g2, src2d, dst2d, zeros128)
    out = _k3(p2, g2, degp_t, bcat)
    return (out[:N, :64], out[:N, 64:])


# speedup vs baseline: 1.2696x; 1.2398x over previous
import functools

import jax
import jax.numpy as jnp
from jax import lax
from jax.experimental import pallas as pl
from jax.experimental.pallas import tpu as pltpu
from jax.experimental.pallas import tpu_sc as plsc

N = 10000
NPAD = 10240
E = 320000
EPAD = 327680
NB = 2560
NC = 2
NS = 16
NW = NC * NS
BPW = NB // NW
RPS = NPAD // NS
HB = NPAD // 128

_mesh = plsc.VectorSubcoreMesh(core_axis_name="c", subcore_axis_name="s")


@functools.partial(
    pl.kernel,
    out_type=jax.ShapeDtypeStruct((NC, HB, 128), jnp.float32),
    mesh=_mesh,
    scratch_types=[
        pltpu.VMEM((BPW, 128), jnp.int32),
        pltpu.VMEM((HB, 128), jnp.float32),
        pltpu.VMEM((HB,), jnp.int32),
        pltpu.VMEM_SHARED((HB, 128), jnp.float32),
    ],
    compiler_params=pltpu.CompilerParams(needs_layout_passes=False),
)
def _deg_pass(dst_hbm, zeros_hbm, idlist_hbm, out_hbm,
              dst_idx, hist, idlist, acc):
    c = lax.axis_index("c")
    s = lax.axis_index("s")
    w = c * NS + s
    pltpu.sync_copy(dst_hbm.at[pl.ds(w * BPW, BPW)], dst_idx)
    pltpu.sync_copy(idlist_hbm, idlist)

    def zbody(i, _):
        for j in range(8):
            hist[i, pl.ds(j * 16, 16)] = jnp.zeros((16,), jnp.float32)
        return 0

    lax.fori_loop(0, HB, zbody, 0)

    @pl.when(s < 10)
    def _():
        pltpu.sync_copy(zeros_hbm, acc.at[pl.ds(s * 8, 8)])

    plsc.subcore_barrier()

    def body(b, _):
        for j in range(8):
            iv = dst_idx[b, pl.ds(j * 16, 16)]
            hi = lax.shift_right_logical(iv, 7)
            lo = lax.bitwise_and(iv, 127)
            cnt, last = plsc.scan_count(iv)
            cur = plsc.load_gather(hist, (hi, lo))
            plsc.store_scatter(hist, (hi, lo),
                               cur + cnt.astype(jnp.float32), mask=last)
        return 0

    lax.fori_loop(0, BPW, body, 0)

    pltpu.sync_copy(hist, acc.at[idlist], add=True)
    plsc.subcore_barrier()

    @pl.when(s < 10)
    def _():
        pltpu.sync_copy(acc.at[pl.ds(s * 8, 8)],
                        out_hbm.at[c, pl.ds(s * 8, 8)])


@functools.partial(
    pl.kernel,
    out_type=jax.ShapeDtypeStruct((NC, NPAD, 128), jnp.float32),
    mesh=_mesh,
    scratch_types=[
        pltpu.VMEM((BPW // 2, 128), jnp.int32),
        pltpu.VMEM((BPW // 2, 128), jnp.int32),
        pltpu.VMEM((128, 128), jnp.float32),
        pltpu.VMEM((128, 128), jnp.float32),
        pltpu.VMEM_SHARED((NPAD, 128), jnp.float32),
        pltpu.SemaphoreType.DMA,
        pltpu.SemaphoreType.DMA,
    ],
)
def _agg_pass(g_hbm, src_hbm, dst_hbm, zeros_hbm, out_hbm,
              src_idx, dst_idx, rows0, rows1, acc, sem0, sem1):
    c = lax.axis_index("c")
    s = lax.axis_index("s")
    w = c * NS + s
    pltpu.sync_copy(zeros_hbm, acc.at[pl.ds(s * RPS, RPS)])
    plsc.subcore_barrier()

    HALF = BPW // 2
    for h in range(2):
        base = w * BPW + h * HALF
        pltpu.sync_copy(src_hbm.at[pl.ds(base, HALF)], src_idx)
        pltpu.sync_copy(dst_hbm.at[pl.ds(base, HALF)], dst_idx)
        pltpu.async_copy(g_hbm.at[src_idx.at[0]], rows0, sem0)

        def body(b2, _):
            b = 2 * b2
            pltpu.make_async_copy(g_hbm.at[pl.ds(0, 128)], rows0, sem0).wait()
            pltpu.async_copy(g_hbm.at[src_idx.at[b + 1]], rows1, sem1)
            pltpu.sync_copy(rows0, acc.at[dst_idx.at[b]], add=True)
            pltpu.make_async_copy(g_hbm.at[pl.ds(0, 128)], rows1, sem1).wait()

            @pl.when(b2 < HALF // 2 - 1)
            def _():
                pltpu.async_copy(g_hbm.at[src_idx.at[b + 2]], rows0, sem0)

            pltpu.sync_copy(rows1, acc.at[dst_idx.at[b + 1]], add=True)
            return 0

        lax.fori_loop(0, HALF // 2, body, 0)
    plsc.subcore_barrier()
    pltpu.sync_copy(acc.at[pl.ds(s * RPS, RPS)],
                    out_hbm.at[c, pl.ds(s * RPS, RPS)])


BLK = 512
NSUB = BLK // 128


def _dinv_of(degp_blk):
    cols = []
    for j in range(NSUB):
        d = degp_blk[j, 0] + degp_blk[j, 1] + 1.0
        cols.append(lax.rsqrt(d)[:, None])
    return jnp.concatenate(cols, axis=0)


def _k1a_body(x_ref, w_ref, h_ref):
    h_ref[...] = jnp.dot(x_ref[...], w_ref[...],
                         preferred_element_type=jnp.float32)


def _k1b_body(h_ref, degp_ref, g_ref):
    g_ref[...] = h_ref[...] * _dinv_of(degp_ref[...])


def _k2_body(p_ref, g1_ref, degp_ref, b1_ref, wcat_ref, g2_ref):
    dinv = _dinv_of(degp_ref[...])
    t = (p_ref[0] + p_ref[1] + g1_ref[...]) * dinv
    h = jnp.maximum(t + b1_ref[...], 0.0)
    g2 = jnp.dot(h, wcat_ref[...], preferred_element_type=jnp.float32) * dinv
    row = pl.program_id(0) * BLK + lax.broadcasted_iota(jnp.int32, (BLK, 1), 0)
    g2_ref[...] = jnp.where(row < N, g2, 0.0)


def _k3_body(p_ref, g2_ref, degp_ref, bcat_ref, o_ref):
    dinv = _dinv_of(degp_ref[...])
    o_ref[...] = (p_ref[0] + p_ref[1] + g2_ref[...]) * dinv + bcat_ref[...]


_spec_rows = pl.BlockSpec((BLK, 128), lambda i: (i, 0))
_spec_w = pl.BlockSpec((128, 128), lambda i: (0, 0))
_spec_degp = pl.BlockSpec((NSUB, 2, 128), lambda i: (i, 0, 0))
_spec_p = pl.BlockSpec((2, BLK, 128), lambda i: (0, i, 0))
_spec_b = pl.BlockSpec((1, 128), lambda i: (0, 0))

_k1a = pl.pallas_call(
    _k1a_body,
    grid=(NPAD // BLK,),
    in_specs=[_spec_rows, _spec_w],
    out_specs=_spec_rows,
    out_shape=jax.ShapeDtypeStruct((NPAD, 128), jnp.float32),
)

_k1b = pl.pallas_call(
    _k1b_body,
    grid=(NPAD // BLK,),
    in_specs=[_spec_rows, _spec_degp],
    out_specs=_spec_rows,
    out_shape=jax.ShapeDtypeStruct((NPAD, 128), jnp.float32),
)

_k2 = pl.pallas_call(
    _k2_body,
    grid=(NPAD // BLK,),
    in_specs=[_spec_p, _spec_rows, _spec_degp, _spec_b, _spec_w],
    out_specs=_spec_rows,
    out_shape=jax.ShapeDtypeStruct((NPAD, 128), jnp.float32),
)

_k3 = pl.pallas_call(
    _k3_body,
    grid=(NPAD // BLK,),
    in_specs=[_spec_p, _spec_rows, _spec_degp, _spec_b],
    out_specs=_spec_rows,
    out_shape=jax.ShapeDtypeStruct((NPAD, 128), jnp.float32),
)


def kernel(x, edge_index, W1, b1, Wmu, bmu, Wls, bls):
    e = edge_index.astype(jnp.int32)
    pad = N + jnp.arange(EPAD - E, dtype=jnp.int32) % (NPAD - N)
    src2d = jnp.concatenate([e[0], pad]).reshape(NB, 128)
    dst2d = jnp.concatenate([e[1], pad]).reshape(NB, 128)

    x_pad = jnp.pad(x, ((0, NPAD - N), (0, 0)))
    wcat = jnp.concatenate([Wmu, Wls], axis=1)
    bcat = jnp.concatenate([bmu, bls]).reshape(1, 128)
    b1r = b1.reshape(1, 128)

    zeros8 = jnp.zeros((8, 128), jnp.float32)
    idlist = jnp.arange(HB, dtype=jnp.int32)
    zeros128 = jnp.zeros((RPS, 128), jnp.float32)

    h1 = _k1a(x_pad, W1)
    degp = _deg_pass(dst2d, zeros8, idlist)
    degp_t = jnp.swapaxes(degp, 0, 1)
    g1 = _k1b(h1, degp_t)
    p1 = _agg_pass(g1, src2d, dst2d, zeros128)
    g2 = _k2(p1, g1, degp_t, b1r, wcat)
    p2 = _agg_pass(g2, src2d, dst2d, zeros128)
    out = _k3(p2, g2, degp_t, bcat)
    return (out[:N, :64], out[:N, 64:])
